# baseline (speedup 1.0000x reference)
import functools

import jax
import jax.numpy as jnp
from jax import lax
from jax.experimental import pallas as pl
from jax.experimental.pallas import tpu as pltpu
from jax.experimental.pallas import tpu_sc as plsc

N_TRAIN = 1000000
EMBED_DIMS = 64
BATCH = 16384

_NC = 2
_NS = 16
_NW = _NC * _NS
_B_PER_W = BATCH // _NW
_L = 16
_G = 1
_NG = _B_PER_W // _G
_NB = 12
_BLK = 128
_GPB = _BLK // _G


def _rsqrt_newton(z):
    i = lax.bitcast_convert_type(z, jnp.int32)
    i = jnp.int32(0x5F3759DF) - lax.shift_right_arithmetic(i, 1)
    y = lax.bitcast_convert_type(i, jnp.float32)
    hz = z * jnp.float32(0.5)
    for _ in range(3):
        y = y * (jnp.float32(1.5) - hz * y * y)
    return y


def _lane_take(x, idx):
    dnums = lax.GatherDimensionNumbers(
        offset_dims=(), collapsed_slice_dims=(0,), start_index_map=(0,))
    return lax.gather(x, idx[:, None], dnums, (1,),
                      mode=lax.GatherScatterMode.PROMISE_IN_BOUNDS)


_mesh = plsc.VectorSubcoreMesh(core_axis_name="c", subcore_axis_name="s")


@functools.partial(
    pl.kernel,
    mesh=_mesh,
    compiler_params=pltpu.CompilerParams(needs_layout_passes=False),
    out_type=jax.ShapeDtypeStruct((EMBED_DIMS, BATCH), jnp.float32),
    scratch_types=[
        pltpu.VMEM((_B_PER_W + _L,), jnp.int32),
        pltpu.VMEM((_NB, _G, EMBED_DIMS, 128), jnp.float32),
        pltpu.VMEM((EMBED_DIMS, _BLK), jnp.float32),
    ] + [pltpu.SemaphoreType.DMA] * _NB,
)
def _embed_norm_t(tab_hbm, idx_hbm, out_hbm, idx_v, stage, outblk, *sems):
    wid = lax.axis_index("s") * _NC + lax.axis_index("c")
    base = wid * _B_PER_W
    pltpu.sync_copy(idx_hbm.at[pl.ds(base, _B_PER_W)],
                    idx_v.at[pl.ds(0, _B_PER_W)])

    lanes = lax.iota(jnp.int32, _L)
    rows = [lanes + jnp.int32(t * _L) for t in range(4)]

    def fire(g, b):
        gvec = idx_v[pl.ds(g * _G, _L)]
        for k in range(_G):
            s_k = gvec[k]
            start = s_k - lax.bitwise_and(s_k, jnp.int32(127))
            start = pl.multiple_of(start, 128)
            pltpu.async_copy(
                tab_hbm.at[:, pl.ds(start, 128)], stage.at[b, k], sems[b])

    def drain(b):
        for k in range(_G):
            pltpu.make_async_copy(
                tab_hbm.at[:, pl.ds(0, 128)], stage.at[b, k], sems[b]).wait()

    def process(g, b):
        gvec = idx_v[pl.ds(g * _G, _L)]
        lvec = lax.bitwise_and(gvec, jnp.int32(127))
        col0 = jnp.full((_L,), lax.rem(g, jnp.int32(_GPB)) * _G,
                        dtype=jnp.int32)
        for k in range(_G):
            lcol = _lane_take(lvec, jnp.full((_L,), k, dtype=jnp.int32))
            vs = [plsc.load_gather(stage.at[b, k], [rows[t], lcol])
                  for t in range(4)]
            s = vs[0] * vs[0] + vs[1] * vs[1] + vs[2] * vs[2] + vs[3] * vs[3]
            for sh in (8, 4, 2, 1):
                s = s + _lane_take(s, lanes ^ sh)
            inv = _rsqrt_newton(jnp.maximum(s, jnp.float32(1e-16)))
            ocol = col0 + k
            for t in range(4):
                plsc.store_scatter(outblk, [rows[t], ocol], vs[t] * inv)

        @pl.when(lax.rem(g, jnp.int32(_GPB)) == _GPB - 1)
        def _():
            t = lax.div(g, jnp.int32(_GPB))
            pltpu.sync_copy(
                outblk, out_hbm.at[:, pl.ds(base + t * _BLK, _BLK)])

    for u in range(_NB):
        fire(jnp.int32(u), u)

    def ring_body(i, _):
        for u in range(_NB):
            g = _NB * i + u
            drain(u)
            process(lax.rem(g, jnp.int32(_NG)), u)
            fire(lax.rem(g + _NB, jnp.int32(_NG)), u)
        return 0

    lax.fori_loop(0, -(-_NG // _NB), ring_body, 0)
    for u in range(_NB):
        drain(u)


def kernel(indices, table):
    tab_t = table.T
    out_t = _embed_norm_t(tab_t, indices.astype(jnp.int32))
    return out_t.T


# speedup vs baseline: 1.0457x; 1.0457x over previous
import functools

import jax
import jax.numpy as jnp
from jax import lax
from jax.experimental import pallas as pl
from jax.experimental.pallas import tpu as pltpu
from jax.experimental.pallas import tpu_sc as plsc

N_TRAIN = 1000000
EMBED_DIMS = 64
BATCH = 16384

_NC = 2
_NS = 16
_NW = _NC * _NS
_B_PER_W = BATCH // _NW
_L = 16
_G = 2
_NG = _B_PER_W // _G
_NB = 7
_BLK = 128
_GPB = _BLK // _G


def _rsqrt_newton(z):
    i = lax.bitcast_convert_type(z, jnp.int32)
    i = jnp.int32(0x5F3759DF) - lax.shift_right_arithmetic(i, 1)
    y = lax.bitcast_convert_type(i, jnp.float32)
    hz = z * jnp.float32(0.5)
    for _ in range(3):
        y = y * (jnp.float32(1.5) - hz * y * y)
    return y


def _lane_take(x, idx):
    dnums = lax.GatherDimensionNumbers(
        offset_dims=(), collapsed_slice_dims=(0,), start_index_map=(0,))
    return lax.gather(x, idx[:, None], dnums, (1,),
                      mode=lax.GatherScatterMode.PROMISE_IN_BOUNDS)


_mesh = plsc.VectorSubcoreMesh(core_axis_name="c", subcore_axis_name="s")


@functools.partial(
    pl.kernel,
    mesh=_mesh,
    compiler_params=pltpu.CompilerParams(needs_layout_passes=False),
    out_type=jax.ShapeDtypeStruct((EMBED_DIMS, BATCH), jnp.float32),
    scratch_types=[
        pltpu.VMEM((_B_PER_W + _L,), jnp.int32),
        pltpu.VMEM((_NB, _G, EMBED_DIMS, 128), jnp.float32),
        pltpu.VMEM((EMBED_DIMS, _BLK), jnp.float32),
    ] + [pltpu.SemaphoreType.DMA] * _NB,
)
def _embed_norm_t(tab_hbm, idx_hbm, out_hbm, idx_v, stage, outblk, *sems):
    wid = lax.axis_index("s") * _NC + lax.axis_index("c")
    base = wid * _B_PER_W
    pltpu.sync_copy(idx_hbm.at[pl.ds(base, _B_PER_W)],
                    idx_v.at[pl.ds(0, _B_PER_W)])

    lanes = lax.iota(jnp.int32, _L)
    rows = [lanes + jnp.int32(t * _L) for t in range(4)]

    def fire(g, b):
        gvec = idx_v[pl.ds(g * _G, _L)]
        for k in range(_G):
            s_k = gvec[k]
            start = s_k - lax.bitwise_and(s_k, jnp.int32(127))
            start = pl.multiple_of(start, 128)
            pltpu.async_copy(
                tab_hbm.at[:, pl.ds(start, 128)], stage.at[b, k], sems[b])

    def drain(b):
        for k in range(_G):
            pltpu.make_async_copy(
                tab_hbm.at[:, pl.ds(0, 128)], stage.at[b, k], sems[b]).wait()

    def process(g, b):
        gvec = idx_v[pl.ds(g * _G, _L)]
        lvec = lax.bitwise_and(gvec, jnp.int32(127))
        col0 = jnp.full((_L,), lax.rem(g, jnp.int32(_GPB)) * _G,
                        dtype=jnp.int32)
        for k in range(_G):
            lcol = _lane_take(lvec, jnp.full((_L,), k, dtype=jnp.int32))
            vs = [plsc.load_gather(stage.at[b, k], [rows[t], lcol])
                  for t in range(4)]
            s = vs[0] * vs[0] + vs[1] * vs[1] + vs[2] * vs[2] + vs[3] * vs[3]
            for sh in (8, 4, 2, 1):
                s = s + _lane_take(s, lanes ^ sh)
            inv = _rsqrt_newton(jnp.maximum(s, jnp.float32(1e-16)))
            ocol = col0 + k
            for t in range(4):
                plsc.store_scatter(outblk, [rows[t], ocol], vs[t] * inv)

        @pl.when(lax.rem(g, jnp.int32(_GPB)) == _GPB - 1)
        def _():
            t = lax.div(g, jnp.int32(_GPB))
            pltpu.sync_copy(
                outblk, out_hbm.at[:, pl.ds(base + t * _BLK, _BLK)])

    for u in range(_NB):
        fire(jnp.int32(u), u)

    def ring_body(i, _):
        for u in range(_NB):
            g = _NB * i + u
            drain(u)
            process(lax.rem(g, jnp.int32(_NG)), u)
            fire(lax.rem(g + _NB, jnp.int32(_NG)), u)
        return 0

    lax.fori_loop(0, -(-_NG // _NB), ring_body, 0)
    for u in range(_NB):
        drain(u)


def kernel(indices, table):
    tab_t = table.T
    out_t = _embed_norm_t(tab_t, indices.astype(jnp.int32))
    return out_t.T
